# baseline (speedup 1.0000x reference)
import functools

import jax
import jax.numpy as jnp
from jax import lax
from jax.experimental import pallas as pl
from jax.experimental.pallas import tpu as pltpu
from jax.experimental.pallas import tpu_sc as plsc

_NC = 2
_NS = 16
_LANES = 16
_CHUNK = 125
_NBUF = 10
_ZBLK = 64


_SC_PARAMS = pltpu.CompilerParams(use_tc_tiling_on_sc=False,
                                  needs_layout_passes=False)


def _sc_mesh():
    return plsc.VectorSubcoreMesh(
        core_axis_name="c", subcore_axis_name="s",
        num_cores=_NC, num_subcores=_NS)


def _sc_degree(dst_w, *, n_pad, cw):
    R = n_pad // _NS

    @functools.partial(
        pl.kernel,
        out_type=jax.ShapeDtypeStruct((_NC, n_pad), jnp.float32),
        mesh=_sc_mesh(),
        compiler_params=_SC_PARAMS,
        scratch_types=[
            pltpu.VMEM_SHARED((n_pad,), jnp.float32),
            pltpu.VMEM((cw, _CHUNK), jnp.int32),
            pltpu.VMEM((128,), jnp.float32),
            pltpu.VMEM((R,), jnp.float32),
            pltpu.SemaphoreType.DMA,
        ],
    )
    def k(dst_hbm, deg_hbm, deg_sh, dst_v, ones_v, degf, sem):
        c = lax.axis_index("c")
        s = lax.axis_index("s")
        w = c * _NS + s
        row0 = s * R

        pltpu.sync_copy(dst_hbm.at[w], dst_v)

        fzero = jnp.zeros((_LANES,), jnp.float32)
        fone = jnp.full((_LANES,), 1.0, jnp.float32)

        def ones_body(i, _):
            ones_v[pl.ds(i * _LANES, _LANES)] = fone
            return 0
        lax.fori_loop(0, 128 // _LANES, ones_body, 0)

        def zf_body(i, _):
            degf[pl.ds(i * _LANES, _LANES)] = fzero
            return 0
        lax.fori_loop(0, R // _LANES, zf_body, 0)
        pltpu.sync_copy(degf, deg_sh.at[pl.ds(row0, R)])
        plsc.subcore_barrier()

        def deg_body(g, _):
            descs = [
                pltpu.async_copy(
                    ones_v.at[pl.ds(0, _CHUNK)],
                    deg_sh.at[dst_v.at[_NBUF * g + b]], sem, add=True)
                for b in range(_NBUF)
            ]
            for dsc in descs:
                dsc.wait()
            return 0
        lax.fori_loop(0, cw // _NBUF, deg_body, 0)
        plsc.subcore_barrier()

        pltpu.sync_copy(deg_sh.at[pl.ds(row0, R)], deg_hbm.at[c, pl.ds(row0, R)])

    return k(dst_w)


def _sc_aggregate(hp, src_w, dst_w, *, n_pad, cw):
    R = n_pad // _NS

    @functools.partial(
        pl.kernel,
        out_type=jax.ShapeDtypeStruct((_NC, n_pad, _LANES), jnp.float32),
        mesh=_sc_mesh(),
        compiler_params=_SC_PARAMS,
        scratch_types=[
            pltpu.VMEM_SHARED((n_pad, _LANES), jnp.float32),
            pltpu.VMEM_SHARED((n_pad, _LANES), jnp.float32),
            pltpu.VMEM((cw, _CHUNK), jnp.int32),
            pltpu.VMEM((cw, _CHUNK), jnp.int32),
            [pltpu.VMEM((_CHUNK, _LANES), jnp.float32)
             for _ in range(_NBUF)],
            pltpu.VMEM((_ZBLK, _LANES), jnp.float32),
            pltpu.SemaphoreType.DMA,
            pltpu.SemaphoreType.DMA,
        ],
    )
    def k(hp_hbm, src_hbm, dst_hbm, acc_hbm,
          hp_sh, acc_sh, src_v, dst_v, rows, zerov, gsem, ssem):
        c = lax.axis_index("c")
        s = lax.axis_index("s")
        w = c * _NS + s
        row0 = s * R

        pltpu.sync_copy(src_hbm.at[w], src_v)
        pltpu.sync_copy(dst_hbm.at[w], dst_v)
        pltpu.sync_copy(hp_hbm.at[pl.ds(row0, R)], hp_sh.at[pl.ds(row0, R)])

        fzero = jnp.zeros((_LANES,), jnp.float32)

        def zfill_body(i, _):
            zerov[i] = fzero
            return 0
        lax.fori_loop(0, _ZBLK, zfill_body, 0)

        def zero_acc(i, _):
            pltpu.sync_copy(zerov, acc_sh.at[pl.ds(row0 + i * _ZBLK, _ZBLK)])
            return 0
        lax.fori_loop(0, R // _ZBLK, zero_acc, 0)
        plsc.subcore_barrier()

        def edge_body(g, _):
            gds = [
                pltpu.async_copy(
                    hp_sh.at[src_v.at[_NBUF * g + b]], rows[b], gsem)
                for b in range(_NBUF)
            ]
            sds = []
            for b in range(_NBUF):
                gds[b].wait()
                sds.append(pltpu.async_copy(
                    rows[b], acc_sh.at[dst_v.at[_NBUF * g + b]], ssem,
                    add=True))
            for dsc in sds:
                dsc.wait()
            return 0
        lax.fori_loop(0, cw // _NBUF, edge_body, 0)
        plsc.subcore_barrier()

        pltpu.sync_copy(acc_sh.at[pl.ds(row0, R)], acc_hbm.at[c, pl.ds(row0, R)])

    return k(hp, src_w, dst_w)


def _tc_linear(x, deg_pair, w1, b1, *, n, n_pad, h):
    def body(x_ref, d0_ref, d1_ref, w_ref, b_ref, hp_ref, dis_ref):
        acc = jnp.dot(x_ref[...], w_ref[...],
                      preferred_element_type=jnp.float32) + b_ref[...]
        rows = lax.broadcasted_iota(jnp.int32, (n_pad, h), 0)
        acc = jnp.where(rows < n, acc, 0.0)
        deg = d0_ref[...] + d1_ref[...] + 1.0
        dis = lax.rsqrt(deg)
        dis_ref[...] = dis
        hp_ref[...] = dis * acc

    d = x.shape[1]
    col = pl.BlockSpec((n_pad, 1), lambda i: (0, 0))
    return pl.pallas_call(
        body,
        grid=(1,),
        in_specs=[
            pl.BlockSpec((n_pad, d), lambda i: (0, 0)),
            col, col,
            pl.BlockSpec((d, h), lambda i: (0, 0)),
            pl.BlockSpec((1, h), lambda i: (0, 0)),
        ],
        out_specs=[pl.BlockSpec((n_pad, h), lambda i: (0, 0)), col],
        out_shape=[jax.ShapeDtypeStruct((n_pad, h), jnp.float32),
                   jax.ShapeDtypeStruct((n_pad, 1), jnp.float32)],
    )(x, deg_pair[0].reshape(n_pad, 1), deg_pair[1].reshape(n_pad, 1),
      w1, b1.reshape(1, h))


def _tc_head(a0, a1, hp, dis, w2, b2, *, n, h, c):
    n8 = -(-n // 8) * 8

    def body(a_ref, b_ref, hp_ref, dis_ref, w_ref, bias_ref, o_ref):
        z = dis_ref[...] * (a_ref[...] + b_ref[...] + hp_ref[...])
        z = jnp.maximum(z, 0.0)
        logits = jnp.dot(z, w_ref[...],
                         preferred_element_type=jnp.float32) + bias_ref[...]
        m = jnp.max(logits, axis=1, keepdims=True)
        e = jnp.exp(logits - m)
        o_ref[...] = e / jnp.sum(e, axis=1, keepdims=True)

    rows = pl.BlockSpec((n8, h), lambda i: (0, 0))
    return pl.pallas_call(
        body,
        grid=(1,),
        in_specs=[
            rows, rows, rows,
            pl.BlockSpec((n8, 1), lambda i: (0, 0)),
            pl.BlockSpec((h, c), lambda i: (0, 0)),
            pl.BlockSpec((1, c), lambda i: (0, 0)),
        ],
        out_specs=pl.BlockSpec((n8, c), lambda i: (0, 0)),
        out_shape=jax.ShapeDtypeStruct((n, c), jnp.float32),
    )(a0, a1, hp, dis, w2, b2.reshape(1, c))


def kernel(x, edge_index, W1, b1, W2, b2):
    n, d = x.shape
    h = W1.shape[1]
    c = W2.shape[1]
    e = edge_index.shape[1]

    n_pad = -(-(n + 64) // 256) * 256
    junk = n_pad - n
    epw = _NC * _NS * _CHUNK * _NBUF
    e_pad = -(-e // epw) * epw
    cw = e_pad // (_NC * _NS * _CHUNK)


    src = edge_index[0]
    dst = edge_index[1]
    pad_cnt = e_pad - e
    if pad_cnt:
        pad_idx = n + jnp.arange(pad_cnt, dtype=jnp.int32) % junk
        src = jnp.concatenate([src, pad_idx])
        dst = jnp.concatenate([dst, pad_idx])
    src_w = src.reshape(_NC * _NS, cw, _CHUNK)
    dst_w = dst.reshape(_NC * _NS, cw, _CHUNK)

    deg_pair = _sc_degree(dst_w, n_pad=n_pad, cw=cw)
    hp, dis = _tc_linear(x, deg_pair, W1, b1, n=n, n_pad=n_pad, h=h)
    acc = _sc_aggregate(hp, src_w, dst_w, n_pad=n_pad, cw=cw)
    return _tc_head(acc[0], acc[1], hp, dis, W2, b2, n=n, h=h, c=c)
